# baseline (speedup 1.0000x reference)
import functools

import jax
import jax.numpy as jnp
from jax.experimental import pallas as pl
from jax.experimental.pallas import tpu as pltpu


def _se_phased_kernel(x_hbm, w1t_ref, b1_ref, a1_ref,
                      w2t_ref, b2_ref, a2_ref,
                      w3t_ref, b3_ref,
                      out_hbm,
                      bufs, rsems, wsems,
                      *, tb, n_groups, n_sub, t_hw, inv_hw):
    def read(g, b):
        for j in range(n_sub):
            pltpu.make_async_copy(
                x_hbm.at[pl.ds(g * tb, tb), :, pl.ds(j * t_hw, t_hw)],
                bufs.at[b, :, :, pl.ds(j * t_hw, t_hw)],
                rsems.at[b, j]).start()

    def wait_read(b):
        for j in range(n_sub):
            pltpu.make_async_copy(
                x_hbm.at[pl.ds(0, tb), :, pl.ds(0, t_hw)],
                bufs.at[b, :, :, pl.ds(0, t_hw)],
                rsems.at[b, j]).wait()

    def write(g, b):
        pltpu.make_async_copy(
            bufs.at[b], out_hbm.at[pl.ds(g * tb, tb)], wsems.at[b]).start()

    def wait_write(b):
        pltpu.make_async_copy(
            bufs.at[b], out_hbm.at[pl.ds(0, tb)], wsems.at[b]).wait()

    w1t = w1t_ref[...]
    w2t = w2t_ref[...]
    w3t = w3t_ref[...]
    b1 = b1_ref[...]
    b2 = b2_ref[...]
    b3 = b3_ref[...]
    a1 = a1_ref[...]
    a2 = a2_ref[...]

    def compute(b):
        x = bufs[b]
        gap = jnp.sum(x, axis=-1) * inv_hw
        y = jnp.dot(gap, w1t, preferred_element_type=jnp.float32) + b1
        y = jnp.where(y >= 0.0, y, a1 * y)
        y = jnp.dot(y, w2t, preferred_element_type=jnp.float32) + b2
        y = jnp.where(y >= 0.0, y, a2 * y)
        y = jnp.dot(y, w3t, preferred_element_type=jnp.float32) + b3
        gate = jax.nn.sigmoid(y).astype(x.dtype)
        bufs[b] = x * gate[:, :, None]

    read(0, 0)
    wait_read(0)
    if n_groups > 1:
        read(1, 1)
    compute(0)
    for g in range(1, n_groups):
        b = g % 2
        wait_read(b)
        write(g - 1, 1 - b)
        compute(b)
        wait_write(1 - b)
        if g + 1 < n_groups:
            read(g + 1, 1 - b)
    write(n_groups - 1, (n_groups - 1) % 2)
    wait_write((n_groups - 1) % 2)


@jax.jit
def kernel(x, w1, b1, a1, w2, b2, a2, w3, b3):
    B, C, H, W = x.shape
    hw = H * W

    n_groups = 4
    while B % n_groups:
        n_groups //= 2
    tb = B // n_groups
    t_hw = 128 if hw % 128 == 0 else hw
    n_sub = hw // t_hw

    x_flat = x.reshape(B, C, hw)

    w1t = w1.T
    w2t = w2.T
    w3t = w3.T

    vmem = lambda shape: pl.BlockSpec(shape, lambda: tuple(0 for _ in shape))
    any_spec = pl.BlockSpec(memory_space=pl.ANY)

    buf_bytes = 2 * tb * C * hw * 4
    weight_bytes = 3 * C * C * 4 + 5 * C * 4
    vmem_limit = int(min(100 * 2**20, buf_bytes + 2 * weight_bytes + 2**20))

    body = functools.partial(
        _se_phased_kernel,
        tb=tb, n_groups=n_groups, n_sub=n_sub, t_hw=t_hw,
        inv_hw=1.0 / float(hw))

    out = pl.pallas_call(
        body,
        out_shape=jax.ShapeDtypeStruct((B, C, hw), x.dtype),
        in_specs=[
            any_spec,
            vmem((C, C)), vmem((1, C)), vmem((1, C)),
            vmem((C, C)), vmem((1, C)), vmem((1, C)),
            vmem((C, C)), vmem((1, C)),
        ],
        out_specs=any_spec,
        scratch_shapes=[
            pltpu.VMEM((2, tb, C, hw), jnp.float32),
            pltpu.SemaphoreType.DMA((2, hw // t_hw)),
            pltpu.SemaphoreType.DMA((2,)),
        ],
        compiler_params=pltpu.CompilerParams(
            vmem_limit_bytes=vmem_limit,
        ),
    )(
        x_flat,
        w1t, b1, a1,
        w2t, b2, a2,
        w3t, b3,
    )
    return out.reshape(B, C, H, W)
